# Initial kernel scaffold; baseline (speedup 1.0000x reference)
import jax
import jax.numpy as jnp
from jax.experimental import pallas as pl


def kernel(s_state, s_val, c_state, c_val, w_s, w_e, w_c, W_expand, W_b2s, W_comp, W_s2b, g_s, b_s, g_e, b_e, g_c, b_c):
    raise NotImplementedError("write your pallas kernel here")


# speedup vs baseline: 11.2429x; 11.2429x over previous
import jax
import jax.numpy as jnp
import numpy as np
from jax.experimental import pallas as pl
from jax.experimental.pallas import tpu as pltpu

DIM = 768
SEQ = 2048
EXP = 4096
COMP = 512
S_WINDOW = 4
S_SCALE = 0.25
SC_B = 0.2
XSC_B2S = 0.075
XSC_S2B = 0.075
SQRT_DIM = float(np.sqrt(np.float32(DIM)))
NEG_INF = float("-inf")

_CC0 = (((0,), (0,)), ((), ()))
_CC1 = (((1,), (1,)), ((), ()))
_F32 = jnp.float32


def _bf(x):
    return x.astype(jnp.bfloat16)


def _dot(a, b):
    return jnp.dot(_bf(a), _bf(b), preferred_element_type=_F32)


def _dott(a, b, dims):
    return jax.lax.dot_general(_bf(a), _bf(b), dims,
                               preferred_element_type=_F32)


def _ln(x, g, b):
    mu = jnp.mean(x, axis=-1, keepdims=True)
    var = jnp.mean((x - mu) ** 2, axis=-1, keepdims=True)
    return (x - mu) / jnp.sqrt(var + 1e-5) * g + b


def _blk_top4(s):
    m1 = jnp.max(s, axis=-1, keepdims=True)
    r1 = jnp.where(s < m1, s, NEG_INF)
    m2 = jnp.max(r1, axis=-1, keepdims=True)
    r2 = jnp.where(r1 < m2, r1, NEG_INF)
    m3 = jnp.max(r2, axis=-1, keepdims=True)
    r3 = jnp.where(r2 < m3, r2, NEG_INF)
    m4 = jnp.max(r3, axis=-1, keepdims=True)
    c1 = jnp.sum((s == m1).astype(jnp.int32), axis=-1, keepdims=True)
    c2 = c1 + jnp.sum((s == m2).astype(jnp.int32), axis=-1, keepdims=True)
    c3 = c2 + jnp.sum((s == m3).astype(jnp.int32), axis=-1, keepdims=True)
    t2 = jnp.where(c1 >= 2, m1, m2)
    t3 = jnp.where(c1 >= 3, m1, jnp.where(c2 >= 3, m2, m3))
    t4 = jnp.where(c1 >= 4, m1, jnp.where(c2 >= 4, m2, jnp.where(c3 >= 4, m3, m4)))
    return m1, t2, t3, t4


def _top4_stats_blocks(blocks):
    tt = jnp.concatenate(_blk_top4(blocks[0]), axis=1)
    for b in blocks[1:]:
        tb = jnp.concatenate(_blk_top4(b), axis=1)
        tt = jnp.concatenate(_blk_top4(jnp.concatenate([tt, tb], axis=1)), axis=1)
    return tt[:, 3:4], tt[:, 0:1]


def _route(logits, nblk):
    cb = logits.shape[1] // nblk
    blocks = [jax.lax.slice_in_dim(logits, c * cb, (c + 1) * cb, axis=1)
              for c in range(nblk)]
    t, m = _top4_stats_blocks(blocks)
    p = jnp.where(logits >= t, jnp.exp(logits - m), 0.0)
    return p / jnp.sum(p, axis=-1, keepdims=True)


SPROP_BLK = 256


def _sprop_body(sp_ref, vp_ref, w_ref, g_ref, b_ref, os_ref, ov_ref):
    i = pl.program_id(0)
    base = i * SPROP_BLK
    halo = SPROP_BLK + 2 * S_WINDOW
    hs = sp_ref[pl.ds(base, halo), :]
    hv = vp_ref[pl.ds(base, halo), :]
    s = jax.lax.slice_in_dim(hs, S_WINDOW, S_WINDOW + SPROP_BLK, axis=0)
    v = jax.lax.slice_in_dim(hv, S_WINDOW, S_WINDOW + SPROP_BLK, axis=0)
    sc_mat = _dott(s * w_ref[...], hs, _CC1) / SQRT_DIM
    row = jax.lax.broadcasted_iota(jnp.int32, (SPROP_BLK, SPROP_BLK + 2 * S_WINDOW), 0)
    col = jax.lax.broadcasted_iota(jnp.int32, (SPROP_BLK, SPROP_BLK + 2 * S_WINDOW), 1)
    diag = col - row
    idx = base + jax.lax.broadcasted_iota(jnp.int32, (SPROP_BLK, 1), 0)
    ks = list(range(-S_WINDOW, S_WINDOW + 1))
    scores = []
    for k in ks:
        sck = jnp.sum(jnp.where(diag == S_WINDOW + k, sc_mat, 0.0),
                      axis=1, keepdims=True)
        valid = (idx + k >= 0) & (idx + k < SEQ)
        scores.append(jnp.where(valid, sck, NEG_INF))
    m = scores[0]
    for sck in scores[1:]:
        m = jnp.maximum(m, sck)
    ps = [jnp.exp(sck - m) for sck in scores]
    denom = ps[0]
    for pk in ps[1:]:
        denom = denom + pk
    ds = jnp.zeros((SPROP_BLK, DIM), _F32)
    dv = jnp.zeros((SPROP_BLK, DIM), _F32)
    for k, pk in zip(ks, ps):
        sj = jax.lax.slice_in_dim(hs, S_WINDOW + k, S_WINDOW + k + SPROP_BLK, axis=0)
        vj = jax.lax.slice_in_dim(hv, S_WINDOW + k, S_WINDOW + k + SPROP_BLK, axis=0)
        a = _bf(pk / denom).astype(_F32)
        ds = ds + a * _bf(sj).astype(_F32)
        dv = dv + a * _bf(vj).astype(_F32)
    os_ref[...] = jnp.tanh(s + S_SCALE * ds)
    ov_ref[...] = _ln(v + S_SCALE * dv, g_ref[...], b_ref[...])


EXP_CBLK = 1024


def _expand_logits_body(cs_ref, cv_ref, Wb_ref, gc_ref, bc_ref, lg_ref, c2_ref, cv2_ref):
    j = pl.program_id(0)
    c2 = jnp.tanh(cs_ref[...])
    lg_ref[...] = _dot(c2, Wb_ref[...])

    @pl.when(j == 0)
    def _():
        c2_ref[...] = c2
        cv2_ref[...] = _ln(cv_ref[...], gc_ref[...], bc_ref[...])


def _expand_scatter_body(lg_ref, c2_ref, cv2_ref, es_ref, ev_ref):
    j = pl.program_id(0)
    nc = EXP // EXP_CBLK
    blocks = [lg_ref[:, pl.ds(c * EXP_CBLK, EXP_CBLK)] for c in range(nc)]
    t, m = _top4_stats_blocks(blocks)
    denom = jnp.zeros_like(t)
    for c in range(nc):
        pb = jnp.where(blocks[c] >= t, jnp.exp(blocks[c] - m), 0.0)
        denom = denom + jnp.sum(pb, axis=-1, keepdims=True)
    lgb = lg_ref[:, pl.ds(j * EXP_CBLK, EXP_CBLK)]
    rb = jnp.where(lgb >= t, jnp.exp(lgb - m), 0.0) / denom
    es_ref[...] = SC_B * _dott(rb, c2_ref[...], _CC0)
    ev_ref[...] = SC_B * _dott(rb, cv2_ref[...], _CC0)


EPROP_BLK = 256
EPROP_CBLK = 512


def _eprop_body(es_ref, ev_ref, we_ref, ge_ref, be_ref, os_ref, ov_ref):
    i = pl.program_id(0)
    eb = es_ref[pl.ds(i * EPROP_BLK, EPROP_BLK), :]
    q = eb * we_ref[...]
    nc = EXP // EPROP_CBLK
    sblocks = []
    for c in range(nc):
        ec = es_ref[pl.ds(c * EPROP_CBLK, EPROP_CBLK), :]
        sblocks.append(_dott(q, ec, _CC1) / SQRT_DIM)
    t, m = _top4_stats_blocks(sblocks)
    denom = jnp.zeros_like(t)
    for c in range(nc):
        sblocks[c] = jnp.where(sblocks[c] >= t, jnp.exp(sblocks[c] - m), 0.0)
        denom = denom + jnp.sum(sblocks[c], axis=-1, keepdims=True)
    ds = jnp.zeros((EPROP_BLK, DIM), _F32)
    dv = jnp.zeros((EPROP_BLK, DIM), _F32)
    for c in range(nc):
        ds = ds + _dot(sblocks[c], es_ref[pl.ds(c * EPROP_CBLK, EPROP_CBLK), :])
        dv = dv + _dot(sblocks[c], ev_ref[pl.ds(c * EPROP_CBLK, EPROP_CBLK), :])
    ds = ds / denom
    dv = dv / denom
    os_ref[...] = jnp.tanh(eb + SC_B * ds)
    ov_ref[...] = _ln(ev_ref[pl.ds(i * EPROP_BLK, EPROP_BLK), :] + SC_B * dv,
                      ge_ref[...], be_ref[...])


B2S_BLK = 256
SEQ_RBLK = 512


def _b2s_body(esb_ref, evb_ref, W_ref, s1_ref, sv1_ref, gs_ref, bs_ref,
              s2_ref, sv2_ref):
    i = pl.program_id(0)
    nb = pl.num_programs(0)
    eb = esb_ref[...]
    evb = evb_ref[...]
    logits = _dot(eb, W_ref[...])
    r = _route(logits, nblk=4)
    for rb in range(SEQ // SEQ_RBLK):
        rsl = jax.lax.slice_in_dim(r, rb * SEQ_RBLK, (rb + 1) * SEQ_RBLK, axis=1)
        ds = _dott(rsl, eb, _CC0)
        dv = _dott(rsl, evb, _CC0)
        rows = pl.ds(rb * SEQ_RBLK, SEQ_RBLK)

        @pl.when(i == 0)
        def _():
            s2_ref[rows, :] = ds
            sv2_ref[rows, :] = dv

        @pl.when(i > 0)
        def _():
            s2_ref[rows, :] = s2_ref[rows, :] + ds
            sv2_ref[rows, :] = sv2_ref[rows, :] + dv

    @pl.when(i == nb - 1)
    def _():
        for rb in range(SEQ // SEQ_RBLK):
            rows = pl.ds(rb * SEQ_RBLK, SEQ_RBLK)
            s2_ref[rows, :] = jnp.tanh(s1_ref[rows, :] + XSC_B2S * s2_ref[rows, :])
            sv2_ref[rows, :] = _ln(sv1_ref[rows, :] + XSC_B2S * sv2_ref[rows, :],
                                   gs_ref[...], bs_ref[...])


def _comp_body(esb_ref, evb_ref, W_ref, dns_ref, dnv_ref):
    i = pl.program_id(0)
    eb = esb_ref[...]
    logits = _dot(eb, W_ref[...])
    r = _route(logits, nblk=1)
    ds = _dott(r, eb, _CC0)
    dv = _dott(r, evb_ref[...], _CC0)

    @pl.when(i == 0)
    def _():
        dns_ref[...] = ds
        dnv_ref[...] = dv

    @pl.when(i > 0)
    def _():
        dns_ref[...] = dns_ref[...] + ds
        dnv_ref[...] = dnv_ref[...] + dv


def _final_body(s2_ref, sv2_ref, dns_ref, dnv_ref, c2_ref, cv2_ref, Ws_ref,
                wc_ref, gc_ref, bc_ref, ns_ref, nv_ref):
    ds2 = jnp.zeros((COMP, DIM), _F32)
    dv2 = jnp.zeros((COMP, DIM), _F32)
    for rb in range(SEQ // SEQ_RBLK):
        rows = pl.ds(rb * SEQ_RBLK, SEQ_RBLK)
        s2b = s2_ref[rows, :]
        lgb = _dot(s2b, Ws_ref[...])
        rr = _route(lgb, nblk=1)
        ds2 = ds2 + _dott(rr, s2b, _CC0)
        dv2 = dv2 + _dott(rr, sv2_ref[rows, :], _CC0)
    np_s = c2_ref[...] + SC_B * dns_ref[...] + XSC_S2B * ds2
    np_v = cv2_ref[...] + SC_B * dnv_ref[...] + XSC_S2B * dv2
    qn = np_s * wc_ref[...]
    scn = _dott(qn, np_s, _CC1) / SQRT_DIM
    a = _route(scn, nblk=1)
    dsn = _dot(a, np_s)
    dvn = _dot(a, np_v)
    ns_ref[...] = jnp.tanh(np_s + SC_B * dsn)
    nv_ref[...] = _ln(np_v + SC_B * dvn, gc_ref[...], bc_ref[...])


def _sds(*shape):
    return jax.ShapeDtypeStruct(shape, _F32)


def _full(shape):
    return pl.BlockSpec(shape, lambda *_: (0,) * len(shape))


_ARB = pltpu.CompilerParams(dimension_semantics=("arbitrary",))


def kernel(s_state, s_val, c_state, c_val, w_s, w_e, w_c, W_expand, W_b2s,
           W_comp, W_s2b, g_s, b_s, g_e, b_e, g_c, b_c):
    batch = s_state.shape[0]
    s = s_state.reshape(SEQ, DIM)
    sv = s_val.reshape(SEQ, DIM)
    cs = c_state.reshape(COMP, DIM)
    cv = c_val.reshape(COMP, DIM)
    ws = w_s.reshape(1, DIM)
    we = w_e.reshape(1, DIM)
    wc = w_c.reshape(1, DIM)
    gs = g_s.reshape(1, DIM)
    bs = b_s.reshape(1, DIM)
    ge = g_e.reshape(1, DIM)
    be = b_e.reshape(1, DIM)
    gc = g_c.reshape(1, DIM)
    bc = b_c.reshape(1, DIM)

    pad = jnp.zeros((S_WINDOW, DIM), _F32)
    sp = jnp.concatenate([pad, s, pad], axis=0)
    vp = jnp.concatenate([pad, sv, pad], axis=0)
    s1, sv1 = pl.pallas_call(
        _sprop_body,
        grid=(SEQ // SPROP_BLK,),
        in_specs=[
            _full((SEQ + 2 * S_WINDOW, DIM)),
            _full((SEQ + 2 * S_WINDOW, DIM)),
            _full((1, DIM)),
            _full((1, DIM)),
            _full((1, DIM)),
        ],
        out_specs=[
            pl.BlockSpec((SPROP_BLK, DIM), lambda i: (i, 0)),
            pl.BlockSpec((SPROP_BLK, DIM), lambda i: (i, 0)),
        ],
        out_shape=[_sds(SEQ, DIM), _sds(SEQ, DIM)],
        compiler_params=_ARB,
    )(sp, vp, ws, gs, bs)

    n_cblk = EXP // EXP_CBLK
    lg, c2, cv2 = pl.pallas_call(
        _expand_logits_body,
        grid=(n_cblk,),
        in_specs=[
            _full((COMP, DIM)),
            _full((COMP, DIM)),
            pl.BlockSpec((DIM, EXP_CBLK), lambda j: (0, j)),
            _full((1, DIM)),
            _full((1, DIM)),
        ],
        out_specs=[
            pl.BlockSpec((COMP, EXP_CBLK), lambda j: (0, j)),
            _full((COMP, DIM)),
            _full((COMP, DIM)),
        ],
        out_shape=[_sds(COMP, EXP), _sds(COMP, DIM), _sds(COMP, DIM)],
        compiler_params=_ARB,
    )(cs, cv, W_expand, gc, bc)

    es0, ev0 = pl.pallas_call(
        _expand_scatter_body,
        grid=(n_cblk,),
        in_specs=[
            _full((COMP, EXP)),
            _full((COMP, DIM)),
            _full((COMP, DIM)),
        ],
        out_specs=[
            pl.BlockSpec((EXP_CBLK, DIM), lambda j: (j, 0)),
            pl.BlockSpec((EXP_CBLK, DIM), lambda j: (j, 0)),
        ],
        out_shape=[_sds(EXP, DIM), _sds(EXP, DIM)],
        compiler_params=_ARB,
    )(lg, c2, cv2)

    es1, ev1 = pl.pallas_call(
        _eprop_body,
        grid=(EXP // EPROP_BLK,),
        in_specs=[
            _full((EXP, DIM)),
            _full((EXP, DIM)),
            _full((1, DIM)),
            _full((1, DIM)),
            _full((1, DIM)),
        ],
        out_specs=[
            pl.BlockSpec((EPROP_BLK, DIM), lambda i: (i, 0)),
            pl.BlockSpec((EPROP_BLK, DIM), lambda i: (i, 0)),
        ],
        out_shape=[_sds(EXP, DIM), _sds(EXP, DIM)],
        compiler_params=_ARB,
    )(es0, ev0, we, ge, be)

    s2, sv2 = pl.pallas_call(
        _b2s_body,
        grid=(EXP // B2S_BLK,),
        in_specs=[
            pl.BlockSpec((B2S_BLK, DIM), lambda i: (i, 0)),
            pl.BlockSpec((B2S_BLK, DIM), lambda i: (i, 0)),
            _full((DIM, SEQ)),
            _full((SEQ, DIM)),
            _full((SEQ, DIM)),
            _full((1, DIM)),
            _full((1, DIM)),
        ],
        out_specs=[_full((SEQ, DIM)), _full((SEQ, DIM))],
        out_shape=[_sds(SEQ, DIM), _sds(SEQ, DIM)],
        compiler_params=_ARB,
    )(es1, ev1, W_b2s, s1, sv1, gs, bs)

    dns, dnv = pl.pallas_call(
        _comp_body,
        grid=(EXP // B2S_BLK,),
        in_specs=[
            pl.BlockSpec((B2S_BLK, DIM), lambda i: (i, 0)),
            pl.BlockSpec((B2S_BLK, DIM), lambda i: (i, 0)),
            _full((DIM, COMP)),
        ],
        out_specs=[_full((COMP, DIM)), _full((COMP, DIM))],
        out_shape=[_sds(COMP, DIM), _sds(COMP, DIM)],
        compiler_params=_ARB,
    )(es1, ev1, W_comp)

    ns, nv = pl.pallas_call(
        _final_body,
        out_shape=[_sds(COMP, DIM), _sds(COMP, DIM)],
    )(s2, sv2, dns, dnv, c2, cv2, W_s2b, wc, gc, bc)

    return (s2.reshape(batch, SEQ, DIM), sv2.reshape(batch, SEQ, DIM),
            ns.reshape(batch, COMP, DIM), nv.reshape(batch, COMP, DIM))
